# baseline (speedup 1.0000x reference)
import functools

import jax
import jax.numpy as jnp
from jax import lax
from jax.experimental import pallas as pl
from jax.experimental.pallas import tpu as pltpu
from jax.experimental.pallas import tpu_sc as plsc

_LANES = 16
_TILES = 16
_CORES = 2
_C = 128
_NBUF = 2
_HMASK = jnp.int32(-65536)
_RND = jnp.int32(0x8000)


def _spmm_pass(tab, fdum, colv, rowv, valv, gbufs, sbufs, acc,
               gsems, ssems, nch, fh):
    def issue_gather(k, u):
        pltpu.async_copy(tab.at[colv.at[k]], gbufs.at[u], gsems[u])

    def wait_gather(u):
        pltpu.make_async_copy(
            tab.at[pl.ds(0, _C)], gbufs.at[u], gsems[u]).wait()

    def wait_scatter(u):
        pltpu.make_async_copy(
            sbufs.at[u], fdum.at[pl.ds(0, _C)], ssems[u]).wait()

    def chunk_work(k, u):
        wait_gather(u)

        def group_body(g, c2):
            vg = valv[k, pl.ds(g * _LANES, _LANES)]
            for e in range(_LANES):
                sc = vg[e]
                r = g * _LANES + e
                for j in range(fh // 32):
                    v = gbufs[u, r, pl.ds(j * _LANES, _LANES)]
                    lo = plsc.bitcast(v << 16, jnp.float32)
                    hi = plsc.bitcast(v & _HMASK, jnp.float32)
                    sbufs[u, r, pl.ds(2 * j * _LANES, _LANES)] = lo * sc
                    sbufs[u, r, pl.ds((2 * j + 1) * _LANES, _LANES)] = hi * sc
            return c2

        lax.fori_loop(0, _C // _LANES, group_body, 0)

        pltpu.async_copy(sbufs.at[u], acc.at[rowv.at[k]], ssems[u], add=True)

    for u in range(_NBUF):
        issue_gather(u, u)

    def block_body(t, carry):
        k0 = t * _NBUF
        for u in range(_NBUF):
            chunk_work(k0 + u, u)
        for u in range(_NBUF):
            wait_scatter(u)
            issue_gather(k0 + _NBUF + u, u)
        return carry

    lax.fori_loop(0, nch // _NBUF - 1, block_body, 0)

    k0 = nch - _NBUF
    for u in range(_NBUF):
        chunk_work(k0 + u, u)
    for u in range(_NBUF):
        wait_scatter(u)


def _sc_cheb_spmm(cols3, rows3, vals3, xbf, np_pad, fh, nch):
    stripe = np_pad // _TILES
    fp = fh // 2
    mesh = plsc.VectorSubcoreMesh(
        core_axis_name="c", subcore_axis_name="s",
        num_cores=_CORES, num_subcores=_TILES,
    )

    @functools.partial(
        pl.kernel,
        out_type=[
            jax.ShapeDtypeStruct((_CORES * np_pad, fh), jnp.float32),
            jax.ShapeDtypeStruct((_CORES * np_pad, fh), jnp.float32),
            jax.ShapeDtypeStruct((_CORES * np_pad, fp), jnp.int32),
        ],
        mesh=mesh,
        compiler_params=pltpu.CompilerParams(
            use_tc_tiling_on_sc=False, needs_layout_passes=False),
        scratch_types=[
            pltpu.VMEM((nch, _C), jnp.int32),
            pltpu.VMEM((nch, _C), jnp.int32),
            pltpu.VMEM((nch, _C), jnp.float32),
            pltpu.VMEM((_NBUF, _C, fp), jnp.int32),
            pltpu.VMEM((_NBUF, _C, fh), jnp.float32),
            pltpu.VMEM_SHARED((np_pad, fh), jnp.float32),
        ] + [pltpu.SemaphoreType.DMA] * (2 * _NBUF),
    )
    def body(cols_h, rows_h, vals_h, xbf_h, s1_h, s2_h, s1bf_h,
             colv, rowv, valv, gbufs, sbufs, acc, *sems):
        gsems = sems[:_NBUF]
        ssems = sems[_NBUF:]
        c = lax.axis_index("c")
        s = lax.axis_index("s")
        base = s * stripe

        pltpu.sync_copy(cols_h.at[s], colv)
        pltpu.sync_copy(rows_h.at[s], rowv)
        pltpu.sync_copy(vals_h.at[s], valv)

        def zero_acc_stripe():
            def zero_row(i, carry):
                for j in range(fh // _LANES):
                    sbufs[0, i, pl.ds(j * _LANES, _LANES)] = jnp.zeros(
                        (_LANES,), jnp.float32)
                return carry

            lax.fori_loop(0, _C, zero_row, 0)
            for kk in range(stripe // _C):
                pltpu.sync_copy(sbufs.at[0], acc.at[pl.ds(base + kk * _C, _C)])

        off = jnp.full((_LANES,), c * np_pad, jnp.int32)

        def off_body(i, carry):
            for j in range(_C // _LANES):
                sl = pl.ds(j * _LANES, _LANES)
                colv[i, sl] = colv[i, sl] + off
            return carry

        lax.fori_loop(0, nch, off_body, 0)

        zero_acc_stripe()
        plsc.subcore_barrier()

        _spmm_pass(xbf_h, s1_h, colv, rowv, valv, gbufs, sbufs, acc,
                   gsems, ssems, nch, fh)
        plsc.subcore_barrier()

        for kk in range(stripe // _C):
            sl = pl.ds(base + kk * _C, _C)
            gsl = pl.ds(c * np_pad + base + kk * _C, _C)
            pltpu.sync_copy(acc.at[sl], sbufs.at[0])
            pltpu.sync_copy(sbufs.at[0], s1_h.at[gsl])

            def pack_row(i, carry):
                for j in range(fh // 32):
                    a = sbufs[0, i, pl.ds(2 * j * _LANES, _LANES)]
                    bb = sbufs[0, i, pl.ds((2 * j + 1) * _LANES, _LANES)]
                    ai = jax.lax.shift_right_logical(
                        plsc.bitcast(a, jnp.int32) + _RND, 16)
                    bi = (plsc.bitcast(bb, jnp.int32) + _RND) & _HMASK
                    gbufs[0, i, pl.ds(j * _LANES, _LANES)] = ai | bi
                return carry

            lax.fori_loop(0, _C, pack_row, 0)
            pltpu.sync_copy(gbufs.at[0], s1bf_h.at[gsl])
        zero_acc_stripe()
        plsc.subcore_barrier()

        _spmm_pass(s1bf_h, s1_h, colv, rowv, valv, gbufs, sbufs, acc,
                   gsems, ssems, nch, fh)
        plsc.subcore_barrier()

        for kk in range(stripe // _C):
            pltpu.sync_copy(
                acc.at[pl.ds(base + kk * _C, _C)],
                s2_h.at[pl.ds(c * np_pad + base + kk * _C, _C)])

    return body(cols3, rows3, vals3, xbf)


def _combine_body(x_ref, s1_ref, s2_ref, w_ref, b_ref, o_ref):
    acc = jnp.dot(x_ref[0], w_ref[0], preferred_element_type=jnp.float32)
    acc += jnp.dot(x_ref[1], w_ref[1], preferred_element_type=jnp.float32)
    acc += jnp.dot(s1_ref[0], w_ref[2], preferred_element_type=jnp.float32)
    acc += jnp.dot(s1_ref[1], w_ref[3], preferred_element_type=jnp.float32)
    acc += jnp.dot(s2_ref[0], w_ref[4], preferred_element_type=jnp.float32)
    acc += jnp.dot(s2_ref[1], w_ref[5], preferred_element_type=jnp.float32)
    o_ref[...] = acc + b_ref[...]


def _tc_combine(xs3, s1s, s2s, wb, bb, np_pad, fh, outf, bm):
    grid = (np_pad // bm,)
    return pl.pallas_call(
        _combine_body,
        grid=grid,
        in_specs=[
            pl.BlockSpec((2, bm, fh), lambda i: (0, i, 0)),
            pl.BlockSpec((2, bm, fh), lambda i: (0, i, 0)),
            pl.BlockSpec((2, bm, fh), lambda i: (0, i, 0)),
            pl.BlockSpec((6, fh, outf), lambda i: (0, 0, 0)),
            pl.BlockSpec((1, outf), lambda i: (0, 0)),
        ],
        out_specs=pl.BlockSpec((bm, outf), lambda i: (i, 0)),
        out_shape=jax.ShapeDtypeStruct((np_pad, outf), jnp.float32),
    )(xs3, s1s, s2s, wb, bb)


def kernel(x, laplacian_indices, laplacian_values, W, b):
    n, f = x.shape
    e = laplacian_values.shape[0]
    outf = W.shape[0]
    k = W.shape[1] // f
    assert k == 3 and f % 64 == 0
    fh = f // 2

    stripe = -(-n // (_TILES * _C)) * _C
    np_pad = _TILES * stripe
    ecb = _C * _NBUF
    ept = -(-e // (_TILES * ecb)) * ecb
    nch = ept // _C
    ep = _TILES * ept

    rows = jnp.pad(laplacian_indices[0], (0, ep - e)).reshape(_TILES, nch, _C)
    cols = jnp.pad(laplacian_indices[1], (0, ep - e)).reshape(_TILES, nch, _C)
    vals = jnp.pad(laplacian_values, (0, ep - e)).reshape(_TILES, nch, _C)

    xp = jnp.pad(x, ((0, np_pad - n), (0, 0)))
    xs_flat = jnp.concatenate([xp[:, :fh], xp[:, fh:]], axis=0)

    xi = xs_flat.reshape(2 * np_pad, fh // 32, 2, _LANES)
    xt = jnp.swapaxes(xi, 2, 3).reshape(2 * np_pad, fh)
    xbf = jax.lax.bitcast_convert_type(
        xt.astype(jnp.bfloat16).reshape(2 * np_pad, fh // 2, 2), jnp.int32)

    w0 = W[:, 0::3]
    w1 = W[:, 1::3]
    w2 = W[:, 2::3]
    a = (w0 - w2).T
    bt = w1.T
    ct = 2.0 * w2.T
    wb = jnp.stack([a[:fh], a[fh:], bt[:fh], bt[fh:], ct[:fh], ct[fh:]])

    s1_flat, s2_flat, _ = _sc_cheb_spmm(cols, rows, vals, xbf,
                                        np_pad, fh, nch)

    xs3 = xs_flat.reshape(2, np_pad, fh)
    s1s = s1_flat.reshape(2, np_pad, fh)
    s2s = s2_flat.reshape(2, np_pad, fh)

    outp = _tc_combine(xs3, s1s, s2s, wb, b.reshape(1, outf),
                       np_pad, fh, outf, bm=640)
    return outp[:n]


# speedup vs baseline: 1.2078x; 1.2078x over previous
import functools

import jax
import jax.numpy as jnp
from jax import lax
from jax.experimental import pallas as pl
from jax.experimental.pallas import tpu as pltpu
from jax.experimental.pallas import tpu_sc as plsc

_LANES = 16
_TILES = 16
_CORES = 2
_C = 128
_NG = 3
_NS = 2
_UNROLL = 6
_HMASK = jnp.int32(-65536)
_RND = jnp.int32(0x8000)


def _spmm_pass(tab, fdum, colv, rowv, valv, gbufs, sbufs, acc,
               gsems, ssems, nch, fh):
    def issue_gather(k, ug):
        pltpu.async_copy(tab.at[colv.at[k]], gbufs.at[ug], gsems[ug])

    def wait_gather(ug):
        pltpu.make_async_copy(
            tab.at[pl.ds(0, _C)], gbufs.at[ug], gsems[ug]).wait()

    def wait_scatter(us):
        pltpu.make_async_copy(
            sbufs.at[us], fdum.at[pl.ds(0, _C)], ssems[us]).wait()

    def chunk_work(k, ug, us):
        wait_gather(ug)

        @pl.when(k >= _NS)
        def _():
            wait_scatter(us)

        kvec = jnp.full((_LANES,), k, jnp.int32)

        def group_body(g, c2):
            gvec = jnp.full((_LANES,), g * _LANES, jnp.int32)
            for e in range(_LANES):
                sc = plsc.load_gather(valv, [kvec, gvec + e])
                r = g * _LANES + e
                for j in range(fh // 32):
                    v = gbufs[ug, r, pl.ds(j * _LANES, _LANES)]
                    lo = plsc.bitcast(v << 16, jnp.float32)
                    hi = plsc.bitcast(v & _HMASK, jnp.float32)
                    sbufs[us, r, pl.ds(2 * j * _LANES, _LANES)] = lo * sc
                    sbufs[us, r, pl.ds((2 * j + 1) * _LANES, _LANES)] = hi * sc
            return c2

        lax.fori_loop(0, _C // _LANES, group_body, 0)

        pltpu.async_copy(sbufs.at[us], acc.at[rowv.at[k]], ssems[us], add=True)

    for ug in range(_NG):
        issue_gather(ug, ug)

    def block_body(t, carry):
        k0 = t * _UNROLL
        for m in range(_UNROLL):
            k = k0 + m
            chunk_work(k, m % _NG, m % _NS)
            k3 = k + _NG

            @pl.when(k3 < nch)
            def _():
                issue_gather(k3, m % _NG)
        return carry

    lax.fori_loop(0, nch // _UNROLL, block_body, 0)

    for us in range(_NS):
        wait_scatter(us)


def _sc_cheb_spmm(cols3, rows3, vals3, xbf, np_pad, n, fh, nch):
    stripe = -(-n // _TILES)
    ra = _TILES * stripe
    chunks = [_C] * (stripe // _C)
    if stripe % _C:
        chunks.append(stripe % _C)
    offs = [sum(chunks[:i]) for i in range(len(chunks))]
    fp = fh // 2
    mesh = plsc.VectorSubcoreMesh(
        core_axis_name="c", subcore_axis_name="s",
        num_cores=_CORES, num_subcores=_TILES,
    )

    @functools.partial(
        pl.kernel,
        out_type=[
            jax.ShapeDtypeStruct((_CORES * np_pad, fh), jnp.float32),
            jax.ShapeDtypeStruct((_CORES * np_pad, fh), jnp.float32),
            jax.ShapeDtypeStruct((_CORES * np_pad, fp), jnp.int32),
        ],
        mesh=mesh,
        compiler_params=pltpu.CompilerParams(
            use_tc_tiling_on_sc=False, needs_layout_passes=False),
        scratch_types=[
            pltpu.VMEM((nch, _C), jnp.int32),
            pltpu.VMEM((nch, _C), jnp.int32),
            pltpu.VMEM((nch, _C), jnp.float32),
            pltpu.VMEM((_NG, _C, fp), jnp.int32),
            pltpu.VMEM((_NS, _C, fh), jnp.float32),
            pltpu.VMEM_SHARED((ra, fh), jnp.float32),
        ] + [pltpu.SemaphoreType.DMA] * (_NG + _NS),
    )
    def body(cols_h, rows_h, vals_h, xbf_h, s1_h, s2_h, s1bf_h,
             colv, rowv, valv, gbufs, sbufs, acc, *sems):
        gsems = sems[:_NG]
        ssems = sems[_NG:]
        c = lax.axis_index("c")
        s = lax.axis_index("s")
        base = s * stripe

        pltpu.sync_copy(cols_h.at[s], colv)
        pltpu.sync_copy(rows_h.at[s], rowv)
        pltpu.sync_copy(vals_h.at[s], valv)

        def zero_acc_stripe():
            def zero_row(i, carry):
                for j in range(fh // _LANES):
                    sbufs[0, i, pl.ds(j * _LANES, _LANES)] = jnp.zeros(
                        (_LANES,), jnp.float32)
                return carry

            lax.fori_loop(0, _C, zero_row, 0)
            for cw, co in zip(chunks, offs):
                pltpu.sync_copy(sbufs.at[0, pl.ds(0, cw)],
                                acc.at[pl.ds(base + co, cw)])

        off = jnp.full((_LANES,), c * np_pad, jnp.int32)

        def off_body(i, carry):
            for j in range(_C // _LANES):
                sl = pl.ds(j * _LANES, _LANES)
                colv[i, sl] = colv[i, sl] + off
            return carry

        lax.fori_loop(0, nch, off_body, 0)

        zero_acc_stripe()
        plsc.subcore_barrier()

        _spmm_pass(xbf_h, s1_h, colv, rowv, valv, gbufs, sbufs, acc,
                   gsems, ssems, nch, fh)
        plsc.subcore_barrier()

        def pack_row(i, carry):
            for j in range(fh // 32):
                a = sbufs[0, i, pl.ds(2 * j * _LANES, _LANES)]
                bb = sbufs[0, i, pl.ds((2 * j + 1) * _LANES, _LANES)]
                ai = jax.lax.shift_right_logical(
                    plsc.bitcast(a, jnp.int32) + _RND, 16)
                bi = (plsc.bitcast(bb, jnp.int32) + _RND) & _HMASK
                gbufs[0, i, pl.ds(j * _LANES, _LANES)] = ai | bi
            return carry

        for cw, co in zip(chunks, offs):
            sl = pl.ds(base + co, cw)
            gsl = pl.ds(c * np_pad + base + co, cw)
            pltpu.sync_copy(acc.at[sl], sbufs.at[0, pl.ds(0, cw)])
            pltpu.sync_copy(sbufs.at[0, pl.ds(0, cw)], s1_h.at[gsl])
            lax.fori_loop(0, cw, pack_row, 0)
            pltpu.sync_copy(gbufs.at[0, pl.ds(0, cw)], s1bf_h.at[gsl])
        zero_acc_stripe()
        plsc.subcore_barrier()

        _spmm_pass(s1bf_h, s1_h, colv, rowv, valv, gbufs, sbufs, acc,
                   gsems, ssems, nch, fh)
        plsc.subcore_barrier()

        for cw, co in zip(chunks, offs):
            pltpu.sync_copy(
                acc.at[pl.ds(base + co, cw)],
                s2_h.at[pl.ds(c * np_pad + base + co, cw)])

    return body(cols3, rows3, vals3, xbf)


def _combine_body(x_ref, s1_ref, s2_ref, w_ref, b_ref, o_ref):
    acc = jnp.dot(x_ref[0], w_ref[0], preferred_element_type=jnp.float32)
    acc += jnp.dot(x_ref[1], w_ref[1], preferred_element_type=jnp.float32)
    acc += jnp.dot(s1_ref[0], w_ref[2], preferred_element_type=jnp.float32)
    acc += jnp.dot(s1_ref[1], w_ref[3], preferred_element_type=jnp.float32)
    acc += jnp.dot(s2_ref[0], w_ref[4], preferred_element_type=jnp.float32)
    acc += jnp.dot(s2_ref[1], w_ref[5], preferred_element_type=jnp.float32)
    o_ref[...] = acc + b_ref[...]


def _tc_combine(xs3, s1s, s2s, wb, bb, np_pad, fh, outf, bm):
    grid = (np_pad // bm,)
    return pl.pallas_call(
        _combine_body,
        grid=grid,
        in_specs=[
            pl.BlockSpec((2, bm, fh), lambda i: (0, i, 0)),
            pl.BlockSpec((2, bm, fh), lambda i: (0, i, 0)),
            pl.BlockSpec((2, bm, fh), lambda i: (0, i, 0)),
            pl.BlockSpec((6, fh, outf), lambda i: (0, 0, 0)),
            pl.BlockSpec((1, outf), lambda i: (0, 0)),
        ],
        out_specs=pl.BlockSpec((bm, outf), lambda i: (i, 0)),
        out_shape=jax.ShapeDtypeStruct((np_pad, outf), jnp.float32),
    )(xs3, s1s, s2s, wb, bb)


def kernel(x, laplacian_indices, laplacian_values, W, b):
    n, f = x.shape
    e = laplacian_values.shape[0]
    outf = W.shape[0]
    k = W.shape[1] // f
    assert k == 3 and f % 64 == 0
    fh = f // 2

    stripe = -(-n // (_TILES * _C)) * _C
    np_pad = _TILES * stripe
    ecb = _C * _UNROLL
    ept = -(-e // (_TILES * ecb)) * ecb
    nch = ept // _C
    ep = _TILES * ept

    rows = jnp.pad(laplacian_indices[0], (0, ep - e)).reshape(_TILES, nch, _C)
    cols = jnp.pad(laplacian_indices[1], (0, ep - e)).reshape(_TILES, nch, _C)
    vals = jnp.pad(laplacian_values, (0, ep - e)).reshape(_TILES, nch, _C)

    xp = jnp.pad(x, ((0, np_pad - n), (0, 0)))
    xs_flat = jnp.concatenate([xp[:, :fh], xp[:, fh:]], axis=0)

    xi = xs_flat.reshape(2 * np_pad, fh // 32, 2, _LANES)
    xt = jnp.swapaxes(xi, 2, 3).reshape(2 * np_pad, fh)
    xbf = jax.lax.bitcast_convert_type(
        xt.astype(jnp.bfloat16).reshape(2 * np_pad, fh // 2, 2), jnp.int32)

    w0 = W[:, 0::3]
    w1 = W[:, 1::3]
    w2 = W[:, 2::3]
    a = (w0 - w2).T
    bt = w1.T
    ct = 2.0 * w2.T
    wb = jnp.stack([a[:fh], a[fh:], bt[:fh], bt[fh:], ct[:fh], ct[fh:]])

    s1_flat, s2_flat, _ = _sc_cheb_spmm(cols, rows, vals, xbf,
                                        np_pad, n, fh, nch)

    xs3 = xs_flat.reshape(2, np_pad, fh)
    s1s = s1_flat.reshape(2, np_pad, fh)
    s2s = s2_flat.reshape(2, np_pad, fh)

    outp = _tc_combine(xs3, s1s, s2s, wb, b.reshape(1, outf),
                       np_pad, fh, outf, bm=640)
    return outp[:n]
